# baseline (speedup 1.0000x reference)
import math

import jax
import jax.numpy as jnp
from jax.experimental import pallas as pl
from jax.experimental.pallas import tpu as pltpu

_K0 = math.sqrt(2.0 / math.pi)
_K1 = 0.044715 * _K0


def _gelu_block(x_ref, o_ref):
    x = x_ref[...]
    x2 = x * x
    z = x * (_K1 * x2 + _K0)
    hx = 0.5 * x
    o_ref[...] = hx + hx * jnp.tanh(z)


def kernel(x, log_tau, log_blend):
    B, T, D = x.shape
    rows = B * T
    x2 = x.reshape(rows, D)
    block = 2040
    out = pl.pallas_call(
        _gelu_block,
        grid=(pl.cdiv(rows, block),),
        in_specs=[pl.BlockSpec((block, D), lambda i: (i, 0))],
        out_specs=pl.BlockSpec((block, D), lambda i: (i, 0)),
        out_shape=jax.ShapeDtypeStruct((rows, D), x.dtype),
        compiler_params=pltpu.CompilerParams(
            vmem_limit_bytes=100 * 1024 * 1024,
            dimension_semantics=("parallel",),
        ),
    )(x2)
    return out.reshape(B, T, D)


# speedup vs baseline: 1.0061x; 1.0061x over previous
import math

import jax
import jax.numpy as jnp
from jax.experimental import pallas as pl
from jax.experimental.pallas import tpu as pltpu

_K0 = math.sqrt(2.0 / math.pi)
_K1 = 0.044715 * _K0


def _gelu_block(x_ref, o_ref):
    x = x_ref[...]
    x2 = x * x
    z = x * (_K1 * x2 + _K0)
    hx = 0.5 * x
    o_ref[...] = hx + hx * jnp.tanh(z)


def kernel(x, log_tau, log_blend):
    B, T, D = x.shape
    rows = B * T
    x2 = x.reshape(rows, D)
    block = 2016
    out = pl.pallas_call(
        _gelu_block,
        grid=(pl.cdiv(rows, block),),
        in_specs=[pl.BlockSpec((block, D), lambda i: (i, 0))],
        out_specs=pl.BlockSpec((block, D), lambda i: (i, 0)),
        out_shape=jax.ShapeDtypeStruct((rows, D), x.dtype),
        compiler_params=pltpu.CompilerParams(
            vmem_limit_bytes=100 * 1024 * 1024,
        ),
    )(x2)
    return out.reshape(B, T, D)
